# baseline (speedup 1.0000x reference)
import functools

import jax
import jax.numpy as jnp
from jax.experimental import pallas as pl
from jax.experimental.pallas import tpu as pltpu
from jax.experimental.pallas import tpu_sc as plsc
import numpy as np

N = 2048
E = 65536
G = 64
NODE_DIM = 128
HID = 128
HEADS = 4
HD = HID // HEADS
TASKS = 8
_P = jax.lax.Precision.HIGHEST


def _split_hi_lo(a):
    hi = a.astype(jnp.bfloat16)
    lo = (a - hi.astype(jnp.float32)).astype(jnp.bfloat16)
    return hi, lo


def _dot_t3(a, b):
    dn = (((1,), (1,)), ((), ()))
    ah, al = _split_hi_lo(a)
    bh, bl = _split_hi_lo(b)
    f = lambda x, y: jax.lax.dot_general(x, y, dn,
                                         preferred_element_type=jnp.float32)
    return f(ah, bh) + f(ah, bl) + f(al, bh)


def _ln(y, g, b):
    m = jnp.mean(y, axis=-1, keepdims=True)
    yc = y - m
    v = jnp.mean(yc * yc, axis=-1, keepdims=True)
    return yc * jax.lax.rsqrt(v + 1e-5) * g + b


def _batch_onehot(batch_row, n_seg):
    seg_ids = jax.lax.broadcasted_iota(jnp.int32, (n_seg, batch_row.shape[1]), 0)
    return jnp.where(seg_ids == batch_row, 1.0, 0.0).astype(jnp.float32)


def _emb_kernel(x_ref, w_ref, b_ref, g_ref, lb_ref, vn_ref, h_ref, xv_ref):
    y = jnp.dot(x_ref[...], w_ref[...], preferred_element_type=jnp.float32, precision=_P)
    h = jax.nn.relu(_ln(y + b_ref[...], g_ref[...], lb_ref[...]))
    h_ref[...] = h
    xv_ref[...] = h + vn_ref[...]


def _emb_stage(x, p_emb, vn_emb):
    out = pl.pallas_call(
        _emb_kernel,
        out_shape=[
            jax.ShapeDtypeStruct((N, HID), jnp.float32),
            jax.ShapeDtypeStruct((N, HID), jnp.float32),
        ],
    )(x, p_emb["lin"]["w"], p_emb["lin"]["b"].reshape(1, HID),
      p_emb["ln"]["g"].reshape(1, HID), p_emb["ln"]["b"].reshape(1, HID),
      vn_emb.reshape(1, HID))
    return out


_GATHER_WINDOW = 256


def _sc_gather(table, idx2d, n_idx):
    mesh = plsc.VectorSubcoreMesh(core_axis_name="core", subcore_axis_name="subcore")

    @functools.partial(
        pl.kernel,
        out_type=jax.ShapeDtypeStruct((n_idx, table.shape[1]), table.dtype),
        mesh=mesh,
    )
    def k(tab_hbm, i_hbm, o_hbm):
        def body(i_vmem, o_vmem):
            pltpu.sync_copy(tab_hbm.at[i_vmem.at[0]], o_vmem)

        pltpu.emit_pipeline(
            body,
            grid=(n_idx // _GATHER_WINDOW,),
            in_specs=[pl.BlockSpec((1, _GATHER_WINDOW), index_map=lambda i: (0, i))],
            out_specs=[pl.BlockSpec((_GATHER_WINDOW, table.shape[1]),
                                    index_map=lambda i: (i, 0))],
            core_axis_name=("core", "subcore"),
            dimension_semantics=(pltpu.PARALLEL,),
        )(i_hbm, o_hbm)

    return k(table, idx2d)


_SCATTER_WINDOW = 256


def _sc_segment_sum(values, idx2d, zeros):
    mesh = plsc.VectorSubcoreMesh(core_axis_name="core", subcore_axis_name="subcore")
    n_e = values.shape[0]
    rows_per_sub = N // 16

    @functools.partial(
        pl.kernel,
        out_type=jax.ShapeDtypeStruct((2, N, HID), jnp.float32),
        mesh=mesh,
        scratch_types=[pltpu.VMEM_SHARED((N, HID), jnp.float32)],
    )
    def k(v_hbm, i_hbm, z_hbm, o_hbm, acc):
        c = jax.lax.axis_index("core")
        s = jax.lax.axis_index("subcore")
        rows = pl.ds(s * rows_per_sub, rows_per_sub)
        pltpu.sync_copy(z_hbm.at[rows], acc.at[rows])
        plsc.subcore_barrier()

        def body(v_vmem, i_vmem):
            pltpu.sync_copy(v_vmem, acc.at[i_vmem.at[0]], add=True)

        pltpu.emit_pipeline(
            body,
            grid=(n_e // _SCATTER_WINDOW,),
            in_specs=[
                pl.BlockSpec((_SCATTER_WINDOW, HID), index_map=lambda i: (i, 0)),
                pl.BlockSpec((1, _SCATTER_WINDOW), index_map=lambda i: (0, i)),
            ],
            out_specs=[],
            core_axis_name=("core", "subcore"),
            dimension_semantics=(pltpu.PARALLEL,),
        )(v_hbm, i_hbm)
        plsc.subcore_barrier()
        pltpu.sync_copy(acc.at[rows], o_hbm.at[c].at[rows])

    return k(values, idx2d, zeros)


def _edge_kernel(ea_ref, xi_ref, xj_ref,
                 we_ref, be_ref, ge_ref, lbe_ref,
                 wa1_ref, wa2_ref, wa3_ref, ba_ref, ga_ref, lba_ref,
                 w2_ref, b2_ref,
                 wm1_ref, wm2_ref, wm3_ref, bm_ref, gm_ref, lbm_ref,
                 msg_ref):
    ea = ea_ref[...]
    xi = xi_ref[...]
    xj = xj_ref[...]
    ef = jnp.dot(ea, we_ref[...], preferred_element_type=jnp.float32) + be_ref[...]
    ef = jax.nn.relu(_ln(ef, ge_ref[...], lbe_ref[...]))

    pre_a = (jnp.dot(xi, wa1_ref[...], preferred_element_type=jnp.float32)
             + jnp.dot(xj, wa2_ref[...], preferred_element_type=jnp.float32)
             + jnp.dot(ef, wa3_ref[...], preferred_element_type=jnp.float32)
             + ba_ref[...])
    ea_h = jax.nn.relu(_ln(pre_a, ga_ref[...], lba_ref[...]))
    logit = jnp.sum(ea_h * w2_ref[...], axis=-1, keepdims=True) + b2_ref[...]
    gate = jax.nn.sigmoid(logit)

    pre_m = (jnp.dot(xi, wm1_ref[...], preferred_element_type=jnp.float32)
             + jnp.dot(xj, wm2_ref[...], preferred_element_type=jnp.float32)
             + jnp.dot(ef, wm3_ref[...], preferred_element_type=jnp.float32)
             + bm_ref[...])
    msg = jax.nn.relu(_ln(pre_m, gm_ref[...], lbm_ref[...]))
    msg_ref[...] = msg * gate


def _edge_stage(edge_attr, gathered, lp, blk_off, n_e):
    EB = 4096
    nb = n_e // EB
    wa = lp["edge_attn"]["lin1"]["w"]
    wm = lp["msg_mlp"]["lin"]["w"]
    row = lambda a: a.reshape(1, -1)
    blk = lambda shape: pl.BlockSpec(shape, lambda i: (0,) * len(shape))
    in_specs = [
        pl.BlockSpec((EB, 16), lambda i: (i + blk_off, 0)),
        pl.BlockSpec((EB, HID), lambda i: (i + nb, 0)),
        pl.BlockSpec((EB, HID), lambda i: (i, 0)),
        blk((16, HID)), blk((1, HID)), blk((1, HID)), blk((1, HID)),
        blk((HID, HID)), blk((HID, HID)), blk((HID, HID)),
        blk((1, HID)), blk((1, HID)), blk((1, HID)),
        blk((1, HID)), blk((1, 1)),
        blk((HID, HID)), blk((HID, HID)), blk((HID, HID)),
        blk((1, HID)), blk((1, HID)), blk((1, HID)),
    ]
    return pl.pallas_call(
        _edge_kernel,
        grid=(nb,),
        in_specs=in_specs,
        out_specs=pl.BlockSpec((EB, HID), lambda i: (i, 0)),
        out_shape=jax.ShapeDtypeStruct((n_e, HID), jnp.float32),
    )(edge_attr, gathered, gathered,
      lp["edge_mlp"]["lin"]["w"], row(lp["edge_mlp"]["lin"]["b"]),
      row(lp["edge_mlp"]["ln"]["g"]), row(lp["edge_mlp"]["ln"]["b"]),
      wa[:HID], wa[HID:2 * HID], wa[2 * HID:],
      row(lp["edge_attn"]["lin1"]["b"]),
      row(lp["edge_attn"]["ln1"]["g"]), row(lp["edge_attn"]["ln1"]["b"]),
      row(lp["edge_attn"]["lin2"]["w"].reshape(-1)),
      lp["edge_attn"]["lin2"]["b"].reshape(1, 1),
      wm[:HID], wm[HID:2 * HID], wm[2 * HID:],
      row(lp["msg_mlp"]["lin"]["b"]),
      row(lp["msg_mlp"]["ln"]["g"]), row(lp["msg_mlp"]["ln"]["b"]))


def _node_kernel(xv_ref, a0_ref, a1_ref, a2_ref, a3_ref, xres_ref, vnode_ref, batch_ref,
                 wq_ref, bq_ref, wk_ref, bk_ref, wv_ref, bv_ref,
                 wo_ref, bo_ref,
                 wu_ref, bu_ref, gu_ref, lbu_ref,
                 wvm_ref, bvm_ref, gvm_ref, lbvm_ref,
                 xn_ref, vn_ref, xvn_ref):
    xv = xv_ref[...]
    aggr = (a0_ref[...] + a1_ref[...]) + (a2_ref[...] + a3_ref[...])
    z = jnp.concatenate([xv, aggr], axis=-1)
    q = jnp.dot(z, wq_ref[...], preferred_element_type=jnp.float32, precision=_P) + bq_ref[...]
    k = jnp.dot(z, wk_ref[...], preferred_element_type=jnp.float32, precision=_P) + bk_ref[...]
    v = jnp.dot(z, wv_ref[...], preferred_element_type=jnp.float32, precision=_P) + bv_ref[...]
    outs = []
    for h in range(HEADS):
        qh = q[:, h * HD:(h + 1) * HD]
        kh = k[:, h * HD:(h + 1) * HD]
        vh = v[:, h * HD:(h + 1) * HD]
        s = _dot_t3(qh, kh)
        s = s * (1.0 / np.sqrt(HD))
        s = s - jnp.max(s, axis=-1, keepdims=True)
        p = jnp.exp(s)
        p = p / jnp.sum(p, axis=-1, keepdims=True)
        outs.append(jnp.dot(p, vh, preferred_element_type=jnp.float32))
    att = jnp.concatenate(outs, axis=-1)
    att = jnp.dot(att, wo_ref[...], preferred_element_type=jnp.float32, precision=_P) + bo_ref[...]

    u_in = jnp.concatenate([xv, att], axis=-1)
    upd = jnp.dot(u_in, wu_ref[...], preferred_element_type=jnp.float32, precision=_P) + bu_ref[...]
    upd = jax.nn.relu(_ln(upd, gu_ref[...], lbu_ref[...]))

    bo = _batch_onehot(batch_ref[...], G)
    vtemp = jnp.dot(bo, upd, preferred_element_type=jnp.float32, precision=_P)
    vt = jnp.dot(vtemp, wvm_ref[...], preferred_element_type=jnp.float32, precision=_P) + bvm_ref[...]
    vnode_new = vnode_ref[...] + jax.nn.relu(_ln(vt, gvm_ref[...], lbvm_ref[...]))

    x_new = upd + xres_ref[...]
    xn_ref[...] = x_new
    vn_ref[...] = vnode_new
    xvn_ref[...] = x_new + jax.lax.dot_general(
        bo, vnode_new, (((0,), (0,)), ((), ())),
        preferred_element_type=jnp.float32, precision=_P)


def _node_stage(xv, aggr_parts, x_res, vnode, batch2d, lp):
    row = lambda a: a.reshape(1, -1)
    na = lp["node_attn"]
    return pl.pallas_call(
        _node_kernel,
        out_shape=[
            jax.ShapeDtypeStruct((N, HID), jnp.float32),
            jax.ShapeDtypeStruct((G, HID), jnp.float32),
            jax.ShapeDtypeStruct((N, HID), jnp.float32),
        ],
    )(xv, aggr_parts[0][0], aggr_parts[0][1], aggr_parts[1][0], aggr_parts[1][1],
      x_res, vnode, batch2d,
      na["q"]["w"], row(na["q"]["b"]), na["k"]["w"], row(na["k"]["b"]),
      na["v"]["w"], row(na["v"]["b"]), na["out"]["w"], row(na["out"]["b"]),
      lp["upd_mlp"]["lin"]["w"], row(lp["upd_mlp"]["lin"]["b"]),
      row(lp["upd_mlp"]["ln"]["g"]), row(lp["upd_mlp"]["ln"]["b"]),
      lp["virt_mlp"]["lin"]["w"], row(lp["virt_mlp"]["lin"]["b"]),
      row(lp["virt_mlp"]["ln"]["g"]), row(lp["virt_mlp"]["ln"]["b"]))


def _mha_small(x, pq, bq, pk, bk, pv, bv, po, bo):
    q = jnp.dot(x, pq, preferred_element_type=jnp.float32, precision=_P) + bq
    k = jnp.dot(x, pk, preferred_element_type=jnp.float32, precision=_P) + bk
    v = jnp.dot(x, pv, preferred_element_type=jnp.float32, precision=_P) + bv
    outs = []
    for h in range(HEADS):
        qh = q[:, h * HD:(h + 1) * HD]
        kh = k[:, h * HD:(h + 1) * HD]
        vh = v[:, h * HD:(h + 1) * HD]
        s = jax.lax.dot_general(qh, kh, (((1,), (1,)), ((), ())),
                                preferred_element_type=jnp.float32, precision=_P)
        s = s * (1.0 / np.sqrt(HD))
        s = s - jnp.max(s, axis=-1, keepdims=True)
        p = jnp.exp(s)
        p = p / jnp.sum(p, axis=-1, keepdims=True)
        outs.append(jnp.dot(p, vh, preferred_element_type=jnp.float32))
    att = jnp.concatenate(outs, axis=-1)
    return jnp.dot(att, po, preferred_element_type=jnp.float32, precision=_P) + bo


def _head_block(z, p, sl):
    h1 = jnp.dot(z, p["lin1"]["w"], preferred_element_type=jnp.float32, precision=_P) + p["lin1"]["b"]
    h1 = jax.nn.relu(_ln(h1, p["ln1"]["g"], p["ln1"]["b"]))
    h2 = jnp.dot(h1, p["lin2"]["w"], preferred_element_type=jnp.float32, precision=_P) + p["lin2"]["b"]
    h2 = jax.nn.relu(_ln(h2, p["ln2"]["g"], p["ln2"]["b"]))
    return jnp.dot(h2, p["lin3"]["w"], preferred_element_type=jnp.float32, precision=_P) + p["lin3"]["b"]


def _final_kernel(x_ref, vnode_ref, batch_ref, ga_refs, hm_refs, hl_refs,
                  out_ref):
    bo = _batch_onehot(batch_ref[...], G)
    cnt = jnp.sum(bo, axis=-1, keepdims=True)
    xg = jnp.dot(bo, x_ref[...], preferred_element_type=jnp.float32, precision=_P)
    xg = xg / jnp.maximum(cnt, 1.0) + vnode_ref[...]
    att = _mha_small(xg, *(r[...] for r in ga_refs))
    pooled = jnp.concatenate([att, xg], axis=-1)

    def tree(refs):
        return {
            "lin1": {"w": refs[0], "b": refs[1]},
            "ln1": {"g": refs[2], "b": refs[3]},
            "lin2": {"w": refs[4], "b": refs[5]},
            "ln2": {"g": refs[6], "b": refs[7]},
            "lin3": {"w": refs[8], "b": refs[9]},
        }

    mean = _head_block(pooled, jax.tree.map(lambda r: r[...], tree(hm_refs)), None)
    logvar = _head_block(pooled, jax.tree.map(lambda r: r[...], tree(hl_refs)), None)
    out_ref[...] = jnp.concatenate([mean, logvar], axis=-1)


def _final_stage(x, vnode, batch2d, params):
    row = lambda a: a.reshape(1, -1)
    ga = params["global_attn"]
    ga_args = (ga["q"]["w"], row(ga["q"]["b"]), ga["k"]["w"], row(ga["k"]["b"]),
               ga["v"]["w"], row(ga["v"]["b"]), ga["out"]["w"], row(ga["out"]["b"]))

    def head_args(p):
        return (p["lin1"]["w"], row(p["lin1"]["b"]), row(p["ln1"]["g"]),
                row(p["ln1"]["b"]), p["lin2"]["w"], row(p["lin2"]["b"]),
                row(p["ln2"]["g"]), row(p["ln2"]["b"]), p["lin3"]["w"],
                row(p["lin3"]["b"]))

    hm = head_args(params["out_mean"])
    hl = head_args(params["out_logvar"])

    def body(x_ref, vnode_ref, batch_ref, *rest):
        ga_refs = rest[:8]
        hm_refs = rest[8:18]
        hl_refs = rest[18:28]
        out_ref = rest[28]
        _final_kernel(x_ref, vnode_ref, batch_ref, ga_refs, hm_refs, hl_refs,
                      out_ref)

    return pl.pallas_call(
        body,
        out_shape=jax.ShapeDtypeStruct((G, 2 * TASKS), jnp.float32),
    )(x, vnode, batch2d, *ga_args, *hm, *hl)


def kernel(x, edge_attr, params, edge_index, batch):
    batch2d = batch.reshape(1, N)
    src = edge_index[0]
    dst = edge_index[1]
    idx2d = edge_index.reshape(1, 2 * E)
    dst2d = edge_index[1:2]
    zeros_nh = jnp.zeros((N, HID), jnp.float32)

    h, xv = _emb_stage(x, params["node_emb"], params["vn_emb"])
    x_cur = h
    vnode = jnp.broadcast_to(params["vn_emb"], (G, HID))

    E2 = E // 2
    nb2 = E2 // 4096
    idx_halves = [
        jnp.concatenate([src[h * E2:(h + 1) * E2],
                         dst[h * E2:(h + 1) * E2]]).reshape(1, 2 * E2)
        for h in range(2)
    ]
    dst_halves = [dst2d[:, h * E2:(h + 1) * E2] for h in range(2)]
    for lp in params["layers"]:
        aggr_parts = []
        g0 = _sc_gather(xv, idx_halves[0], 2 * E2)
        g1 = _sc_gather(xv, idx_halves[1], 2 * E2)
        msg0 = _edge_stage(edge_attr, g0, lp, 0, E2)
        aggr_parts.append(_sc_segment_sum(msg0, dst_halves[0], zeros_nh))
        msg1 = _edge_stage(edge_attr, g1, lp, nb2, E2)
        aggr_parts.append(_sc_segment_sum(msg1, dst_halves[1], zeros_nh))
        x_cur, vnode, xv = _node_stage(xv, aggr_parts, x_cur, vnode, batch2d, lp)

    return _final_stage(x_cur, vnode, batch2d, params)


# speedup vs baseline: 1.0087x; 1.0087x over previous
import functools

import jax
import jax.numpy as jnp
from jax.experimental import pallas as pl
from jax.experimental.pallas import tpu as pltpu
from jax.experimental.pallas import tpu_sc as plsc
import numpy as np

N = 2048
E = 65536
G = 64
NODE_DIM = 128
HID = 128
HEADS = 4
HD = HID // HEADS
TASKS = 8
_P = jax.lax.Precision.HIGHEST


def _split_hi_lo(a):
    hi = a.astype(jnp.bfloat16)
    lo = (a - hi.astype(jnp.float32)).astype(jnp.bfloat16)
    return hi, lo


def _dot_t3(a, b):
    dn = (((1,), (1,)), ((), ()))
    ah, al = _split_hi_lo(a)
    bh, bl = _split_hi_lo(b)
    f = lambda x, y: jax.lax.dot_general(x, y, dn,
                                         preferred_element_type=jnp.float32)
    return f(ah, bh) + f(ah, bl) + f(al, bh)


def _ln(y, g, b):
    m = jnp.mean(y, axis=-1, keepdims=True)
    yc = y - m
    v = jnp.mean(yc * yc, axis=-1, keepdims=True)
    return yc * jax.lax.rsqrt(v + 1e-5) * g + b


def _batch_onehot(batch_row, n_seg):
    seg_ids = jax.lax.broadcasted_iota(jnp.int32, (n_seg, batch_row.shape[1]), 0)
    return jnp.where(seg_ids == batch_row, 1.0, 0.0).astype(jnp.float32)


def _emb_kernel(x_ref, w_ref, b_ref, g_ref, lb_ref, vn_ref, h_ref, xv_ref):
    y = jnp.dot(x_ref[...], w_ref[...], preferred_element_type=jnp.float32, precision=_P)
    h = jax.nn.relu(_ln(y + b_ref[...], g_ref[...], lb_ref[...]))
    h_ref[...] = h
    xv_ref[...] = h + vn_ref[...]


def _emb_stage(x, p_emb, vn_emb):
    out = pl.pallas_call(
        _emb_kernel,
        out_shape=[
            jax.ShapeDtypeStruct((N, HID), jnp.float32),
            jax.ShapeDtypeStruct((N, HID), jnp.float32),
        ],
    )(x, p_emb["lin"]["w"], p_emb["lin"]["b"].reshape(1, HID),
      p_emb["ln"]["g"].reshape(1, HID), p_emb["ln"]["b"].reshape(1, HID),
      vn_emb.reshape(1, HID))
    return out


_GATHER_WINDOW = 128


def _sc_gather(table, idx2d, n_idx):
    mesh = plsc.VectorSubcoreMesh(core_axis_name="core", subcore_axis_name="subcore")

    @functools.partial(
        pl.kernel,
        out_type=jax.ShapeDtypeStruct((n_idx, table.shape[1]), table.dtype),
        mesh=mesh,
    )
    def k(tab_hbm, i_hbm, o_hbm):
        def body(i_vmem, o_vmem):
            pltpu.sync_copy(tab_hbm.at[i_vmem.at[0]], o_vmem)

        pltpu.emit_pipeline(
            body,
            grid=(n_idx // _GATHER_WINDOW,),
            in_specs=[pl.BlockSpec((1, _GATHER_WINDOW), index_map=lambda i: (0, i))],
            out_specs=[pl.BlockSpec((_GATHER_WINDOW, table.shape[1]),
                                    index_map=lambda i: (i, 0))],
            core_axis_name=("core", "subcore"),
            dimension_semantics=(pltpu.PARALLEL,),
        )(i_hbm, o_hbm)

    return k(table, idx2d)


_SCATTER_WINDOW = 256


def _sc_segment_sum(values, idx2d, zeros):
    mesh = plsc.VectorSubcoreMesh(core_axis_name="core", subcore_axis_name="subcore")
    n_e = values.shape[0]
    rows_per_sub = N // 16

    @functools.partial(
        pl.kernel,
        out_type=jax.ShapeDtypeStruct((2, N, HID), jnp.float32),
        mesh=mesh,
        scratch_types=[pltpu.VMEM_SHARED((N, HID), jnp.float32)],
    )
    def k(v_hbm, i_hbm, z_hbm, o_hbm, acc):
        c = jax.lax.axis_index("core")
        s = jax.lax.axis_index("subcore")
        rows = pl.ds(s * rows_per_sub, rows_per_sub)
        pltpu.sync_copy(z_hbm.at[rows], acc.at[rows])
        plsc.subcore_barrier()

        def body(v_vmem, i_vmem):
            pltpu.sync_copy(v_vmem, acc.at[i_vmem.at[0]], add=True)

        pltpu.emit_pipeline(
            body,
            grid=(n_e // _SCATTER_WINDOW,),
            in_specs=[
                pl.BlockSpec((_SCATTER_WINDOW, HID), index_map=lambda i: (i, 0)),
                pl.BlockSpec((1, _SCATTER_WINDOW), index_map=lambda i: (0, i)),
            ],
            out_specs=[],
            core_axis_name=("core", "subcore"),
            dimension_semantics=(pltpu.PARALLEL,),
        )(v_hbm, i_hbm)
        plsc.subcore_barrier()
        pltpu.sync_copy(acc.at[rows], o_hbm.at[c].at[rows])

    return k(values, idx2d, zeros)


def _edge_kernel(ea_ref, xi_ref, xj_ref,
                 we_ref, be_ref, ge_ref, lbe_ref,
                 wa1_ref, wa2_ref, wa3_ref, ba_ref, ga_ref, lba_ref,
                 w2_ref, b2_ref,
                 wm1_ref, wm2_ref, wm3_ref, bm_ref, gm_ref, lbm_ref,
                 msg_ref):
    ea = ea_ref[...]
    xi = xi_ref[...]
    xj = xj_ref[...]
    ef = jnp.dot(ea, we_ref[...], preferred_element_type=jnp.float32) + be_ref[...]
    ef = jax.nn.relu(_ln(ef, ge_ref[...], lbe_ref[...]))

    pre_a = (jnp.dot(xi, wa1_ref[...], preferred_element_type=jnp.float32)
             + jnp.dot(xj, wa2_ref[...], preferred_element_type=jnp.float32)
             + jnp.dot(ef, wa3_ref[...], preferred_element_type=jnp.float32)
             + ba_ref[...])
    ea_h = jax.nn.relu(_ln(pre_a, ga_ref[...], lba_ref[...]))
    logit = jnp.sum(ea_h * w2_ref[...], axis=-1, keepdims=True) + b2_ref[...]
    gate = jax.nn.sigmoid(logit)

    pre_m = (jnp.dot(xi, wm1_ref[...], preferred_element_type=jnp.float32)
             + jnp.dot(xj, wm2_ref[...], preferred_element_type=jnp.float32)
             + jnp.dot(ef, wm3_ref[...], preferred_element_type=jnp.float32)
             + bm_ref[...])
    msg = jax.nn.relu(_ln(pre_m, gm_ref[...], lbm_ref[...]))
    msg_ref[...] = msg * gate


def _edge_stage(edge_attr, gathered, lp, blk_off, n_e):
    EB = 4096
    nb = n_e // EB
    wa = lp["edge_attn"]["lin1"]["w"]
    wm = lp["msg_mlp"]["lin"]["w"]
    wa1 = wa[:HID]
    wa2 = wa[HID:2 * HID]
    wm1 = wm[:HID]
    wm2 = wm[HID:2 * HID]
    row = lambda a: a.reshape(1, -1)
    blk = lambda shape: pl.BlockSpec(shape, lambda i: (0,) * len(shape))
    in_specs = [
        pl.BlockSpec((EB, 16), lambda i: (i + blk_off, 0)),
        pl.BlockSpec((EB, HID), lambda i: (i + nb, 0)),
        pl.BlockSpec((EB, HID), lambda i: (i, 0)),
        blk((16, HID)), blk((1, HID)), blk((1, HID)), blk((1, HID)),
        blk((HID, HID)), blk((HID, HID)), blk((HID, HID)),
        blk((1, HID)), blk((1, HID)), blk((1, HID)),
        blk((1, HID)), blk((1, 1)),
        blk((HID, HID)), blk((HID, HID)), blk((HID, HID)),
        blk((1, HID)), blk((1, HID)), blk((1, HID)),
    ]
    return pl.pallas_call(
        _edge_kernel,
        grid=(nb,),
        in_specs=in_specs,
        out_specs=pl.BlockSpec((EB, HID), lambda i: (i, 0)),
        out_shape=jax.ShapeDtypeStruct((n_e, HID), jnp.float32),
    )(edge_attr, gathered, gathered,
      lp["edge_mlp"]["lin"]["w"], row(lp["edge_mlp"]["lin"]["b"]),
      row(lp["edge_mlp"]["ln"]["g"]), row(lp["edge_mlp"]["ln"]["b"]),
      wa1, wa2, wa[2 * HID:],
      row(lp["edge_attn"]["lin1"]["b"]),
      row(lp["edge_attn"]["ln1"]["g"]), row(lp["edge_attn"]["ln1"]["b"]),
      row(lp["edge_attn"]["lin2"]["w"].reshape(-1)),
      lp["edge_attn"]["lin2"]["b"].reshape(1, 1),
      wm1, wm2, wm[2 * HID:],
      row(lp["msg_mlp"]["lin"]["b"]),
      row(lp["msg_mlp"]["ln"]["g"]), row(lp["msg_mlp"]["ln"]["b"]))


def _node_kernel(xv_ref, a0_ref, a1_ref, a2_ref, a3_ref, xres_ref, vnode_ref, batch_ref,
                 wq_ref, bq_ref, wk_ref, bk_ref, wv_ref, bv_ref,
                 wo_ref, bo_ref,
                 wu_ref, bu_ref, gu_ref, lbu_ref,
                 wvm_ref, bvm_ref, gvm_ref, lbvm_ref,
                 xn_ref, vn_ref, xvn_ref):
    xv = xv_ref[...]
    aggr = (a0_ref[...] + a1_ref[...]) + (a2_ref[...] + a3_ref[...])
    z = jnp.concatenate([xv, aggr], axis=-1)
    q = jnp.dot(z, wq_ref[...], preferred_element_type=jnp.float32, precision=_P) + bq_ref[...]
    k = jnp.dot(z, wk_ref[...], preferred_element_type=jnp.float32, precision=_P) + bk_ref[...]
    v = jnp.dot(z, wv_ref[...], preferred_element_type=jnp.float32, precision=_P) + bv_ref[...]
    outs = []
    for h in range(HEADS):
        qh = q[:, h * HD:(h + 1) * HD]
        kh = k[:, h * HD:(h + 1) * HD]
        vh = v[:, h * HD:(h + 1) * HD]
        s = _dot_t3(qh, kh)
        s = s * (1.0 / np.sqrt(HD))
        s = s - jnp.max(s, axis=-1, keepdims=True)
        p = jnp.exp(s)
        p = p / jnp.sum(p, axis=-1, keepdims=True)
        outs.append(jnp.dot(p, vh, preferred_element_type=jnp.float32))
    att = jnp.concatenate(outs, axis=-1)
    att = jnp.dot(att, wo_ref[...], preferred_element_type=jnp.float32, precision=_P) + bo_ref[...]

    u_in = jnp.concatenate([xv, att], axis=-1)
    upd = jnp.dot(u_in, wu_ref[...], preferred_element_type=jnp.float32, precision=_P) + bu_ref[...]
    upd = jax.nn.relu(_ln(upd, gu_ref[...], lbu_ref[...]))

    bo = _batch_onehot(batch_ref[...], G)
    vtemp = jnp.dot(bo, upd, preferred_element_type=jnp.float32, precision=_P)
    vt = jnp.dot(vtemp, wvm_ref[...], preferred_element_type=jnp.float32, precision=_P) + bvm_ref[...]
    vnode_new = vnode_ref[...] + jax.nn.relu(_ln(vt, gvm_ref[...], lbvm_ref[...]))

    x_new = upd + xres_ref[...]
    xn_ref[...] = x_new
    vn_ref[...] = vnode_new
    xvn_ref[...] = x_new + jax.lax.dot_general(
        bo, vnode_new, (((0,), (0,)), ((), ())),
        preferred_element_type=jnp.float32, precision=_P)


def _node_stage(xv, aggr_parts, x_res, vnode, batch2d, lp):
    row = lambda a: a.reshape(1, -1)
    na = lp["node_attn"]
    return pl.pallas_call(
        _node_kernel,
        out_shape=[
            jax.ShapeDtypeStruct((N, HID), jnp.float32),
            jax.ShapeDtypeStruct((G, HID), jnp.float32),
            jax.ShapeDtypeStruct((N, HID), jnp.float32),
        ],
    )(xv, aggr_parts[0][0], aggr_parts[0][1], aggr_parts[1][0], aggr_parts[1][1],
      x_res, vnode, batch2d,
      na["q"]["w"], row(na["q"]["b"]), na["k"]["w"], row(na["k"]["b"]),
      na["v"]["w"], row(na["v"]["b"]), na["out"]["w"], row(na["out"]["b"]),
      lp["upd_mlp"]["lin"]["w"], row(lp["upd_mlp"]["lin"]["b"]),
      row(lp["upd_mlp"]["ln"]["g"]), row(lp["upd_mlp"]["ln"]["b"]),
      lp["virt_mlp"]["lin"]["w"], row(lp["virt_mlp"]["lin"]["b"]),
      row(lp["virt_mlp"]["ln"]["g"]), row(lp["virt_mlp"]["ln"]["b"]))


def _mha_small(x, pq, bq, pk, bk, pv, bv, po, bo):
    q = jnp.dot(x, pq, preferred_element_type=jnp.float32, precision=_P) + bq
    k = jnp.dot(x, pk, preferred_element_type=jnp.float32, precision=_P) + bk
    v = jnp.dot(x, pv, preferred_element_type=jnp.float32, precision=_P) + bv
    outs = []
    for h in range(HEADS):
        qh = q[:, h * HD:(h + 1) * HD]
        kh = k[:, h * HD:(h + 1) * HD]
        vh = v[:, h * HD:(h + 1) * HD]
        s = jax.lax.dot_general(qh, kh, (((1,), (1,)), ((), ())),
                                preferred_element_type=jnp.float32, precision=_P)
        s = s * (1.0 / np.sqrt(HD))
        s = s - jnp.max(s, axis=-1, keepdims=True)
        p = jnp.exp(s)
        p = p / jnp.sum(p, axis=-1, keepdims=True)
        outs.append(jnp.dot(p, vh, preferred_element_type=jnp.float32))
    att = jnp.concatenate(outs, axis=-1)
    return jnp.dot(att, po, preferred_element_type=jnp.float32, precision=_P) + bo


def _head_block(z, p, sl):
    h1 = jnp.dot(z, p["lin1"]["w"], preferred_element_type=jnp.float32, precision=_P) + p["lin1"]["b"]
    h1 = jax.nn.relu(_ln(h1, p["ln1"]["g"], p["ln1"]["b"]))
    h2 = jnp.dot(h1, p["lin2"]["w"], preferred_element_type=jnp.float32, precision=_P) + p["lin2"]["b"]
    h2 = jax.nn.relu(_ln(h2, p["ln2"]["g"], p["ln2"]["b"]))
    return jnp.dot(h2, p["lin3"]["w"], preferred_element_type=jnp.float32, precision=_P) + p["lin3"]["b"]


def _final_kernel(x_ref, vnode_ref, batch_ref, ga_refs, hm_refs, hl_refs,
                  out_ref):
    bo = _batch_onehot(batch_ref[...], G)
    cnt = jnp.sum(bo, axis=-1, keepdims=True)
    xg = jnp.dot(bo, x_ref[...], preferred_element_type=jnp.float32, precision=_P)
    xg = xg / jnp.maximum(cnt, 1.0) + vnode_ref[...]
    att = _mha_small(xg, *(r[...] for r in ga_refs))
    pooled = jnp.concatenate([att, xg], axis=-1)

    def tree(refs):
        return {
            "lin1": {"w": refs[0], "b": refs[1]},
            "ln1": {"g": refs[2], "b": refs[3]},
            "lin2": {"w": refs[4], "b": refs[5]},
            "ln2": {"g": refs[6], "b": refs[7]},
            "lin3": {"w": refs[8], "b": refs[9]},
        }

    mean = _head_block(pooled, jax.tree.map(lambda r: r[...], tree(hm_refs)), None)
    logvar = _head_block(pooled, jax.tree.map(lambda r: r[...], tree(hl_refs)), None)
    out_ref[...] = jnp.concatenate([mean, logvar], axis=-1)


def _final_stage(x, vnode, batch2d, params):
    row = lambda a: a.reshape(1, -1)
    ga = params["global_attn"]
    ga_args = (ga["q"]["w"], row(ga["q"]["b"]), ga["k"]["w"], row(ga["k"]["b"]),
               ga["v"]["w"], row(ga["v"]["b"]), ga["out"]["w"], row(ga["out"]["b"]))

    def head_args(p):
        return (p["lin1"]["w"], row(p["lin1"]["b"]), row(p["ln1"]["g"]),
                row(p["ln1"]["b"]), p["lin2"]["w"], row(p["lin2"]["b"]),
                row(p["ln2"]["g"]), row(p["ln2"]["b"]), p["lin3"]["w"],
                row(p["lin3"]["b"]))

    hm = head_args(params["out_mean"])
    hl = head_args(params["out_logvar"])

    def body(x_ref, vnode_ref, batch_ref, *rest):
        ga_refs = rest[:8]
        hm_refs = rest[8:18]
        hl_refs = rest[18:28]
        out_ref = rest[28]
        _final_kernel(x_ref, vnode_ref, batch_ref, ga_refs, hm_refs, hl_refs,
                      out_ref)

    return pl.pallas_call(
        body,
        out_shape=jax.ShapeDtypeStruct((G, 2 * TASKS), jnp.float32),
    )(x, vnode, batch2d, *ga_args, *hm, *hl)


def kernel(x, edge_attr, params, edge_index, batch):
    batch2d = batch.reshape(1, N)
    src = edge_index[0]
    dst = edge_index[1]
    idx2d = edge_index.reshape(1, 2 * E)
    dst2d = edge_index[1:2]
    zeros_nh = jnp.zeros((N, HID), jnp.float32)

    h, xv = _emb_stage(x, params["node_emb"], params["vn_emb"])
    x_cur = h
    vnode = jnp.broadcast_to(params["vn_emb"], (G, HID))

    E2 = E // 2
    nb2 = E2 // 4096
    idx_halves = [
        jnp.concatenate([src[h * E2:(h + 1) * E2],
                         dst[h * E2:(h + 1) * E2]]).reshape(1, 2 * E2)
        for h in range(2)
    ]
    dst_halves = [dst2d[:, h * E2:(h + 1) * E2] for h in range(2)]
    for lp in params["layers"]:
        aggr_parts = []
        g0 = _sc_gather(xv, idx_halves[0], 2 * E2)
        g1 = _sc_gather(xv, idx_halves[1], 2 * E2)
        msg0 = _edge_stage(edge_attr, g0, lp, 0, E2)
        aggr_parts.append(_sc_segment_sum(msg0, dst_halves[0], zeros_nh))
        msg1 = _edge_stage(edge_attr, g1, lp, nb2, E2)
        aggr_parts.append(_sc_segment_sum(msg1, dst_halves[1], zeros_nh))
        x_cur, vnode, xv = _node_stage(xv, aggr_parts, x_cur, vnode, batch2d, lp)

    return _final_stage(x_cur, vnode, batch2d, params)
